# baseline (speedup 1.0000x reference)
import functools
import math

import jax
import jax.numpy as jnp
from jax import lax
from jax.experimental import pallas as pl
from jax.experimental.pallas import tpu as pltpu
from jax.experimental.pallas import tpu_sc as plsc

VOCAB = 1000000
D = 64
B = 4096 * 200
NC, NS, L = 2, 16, 16
NW = NC * NS
B_PER_W = B // NW
CH = 128
N_CH = B_PER_W // CH
SCALE = math.sqrt(D)


def _sc_embed(table, x_flat):
  mesh = plsc.VectorSubcoreMesh(core_axis_name="c", subcore_axis_name="s")

  @functools.partial(
      pl.kernel,
      mesh=mesh,
      compiler_params=pltpu.CompilerParams(use_tc_tiling_on_sc=False),
      out_type=jax.ShapeDtypeStruct((B, D), jnp.float32),
      scratch_types=[
          pltpu.VMEM((CH,), jnp.int32),
          pltpu.VMEM((CH, D), jnp.float32),
          pltpu.SemaphoreType.DMA,
      ],
  )
  def k(table_hbm, idx_hbm, out_hbm, idx_v, rows_v, sem):
    wid = lax.axis_index("s") * NC + lax.axis_index("c")
    base = wid * B_PER_W

    def chunk_body(g, _):
      off = base + g * CH
      pltpu.sync_copy(idx_hbm.at[pl.ds(off, CH)], idx_v)
      pltpu.async_copy(table_hbm.at[idx_v], rows_v, sem).wait()

      def scale_body(r, _):
        for j in range(D // L):
          sl = pl.ds(j * L, L)
          rows_v[r, sl] = rows_v[r, sl] * SCALE
        return 0

      lax.fori_loop(0, CH, scale_body, 0)
      pltpu.sync_copy(rows_v, out_hbm.at[pl.ds(off, CH)])
      return 0

    lax.fori_loop(0, N_CH, chunk_body, 0)

  return k(table, x_flat)


def kernel(table, x):
  x_flat = x.reshape(-1).astype(jnp.int32)
  out = _sc_embed(table, x_flat)
  return out.reshape(x.shape + (D,))


# speedup vs baseline: 1.2732x; 1.2732x over previous
import functools
import math

import jax
import jax.numpy as jnp
from jax import lax
from jax.experimental import pallas as pl
from jax.experimental.pallas import tpu as pltpu
from jax.experimental.pallas import tpu_sc as plsc

VOCAB = 1000000
D = 64
B = 4096 * 200
NC, NS, L = 2, 16, 16
NW = NC * NS
B_PER_W = B // NW
CH = 128
K = 4
SS = K * CH
N_SS = B_PER_W // SS
SCALE = math.sqrt(D)
RU = 4


def _sc_embed(table, x_flat):
  mesh = plsc.VectorSubcoreMesh(core_axis_name="c", subcore_axis_name="s")

  @functools.partial(
      pl.kernel,
      mesh=mesh,
      compiler_params=pltpu.CompilerParams(use_tc_tiling_on_sc=False),
      out_type=jax.ShapeDtypeStruct((B, D), jnp.float32),
      scratch_types=[
          pltpu.VMEM((B_PER_W,), jnp.int32),
          pltpu.VMEM((2, K, CH, D), jnp.float32),
          pltpu.SemaphoreType.DMA((2, K)),
          pltpu.SemaphoreType.DMA((2, K)),
      ],
  )
  def k(table_hbm, idx_hbm, out_hbm, idx_v, rows_v, gsem, osem):
    wid = lax.axis_index("s") * NC + lax.axis_index("c")
    base = wid * B_PER_W

    pltpu.sync_copy(idx_hbm.at[pl.ds(base, B_PER_W)], idx_v)

    def fire(s, buf, first):
      for j in range(K):
        ioff = s * SS + j * CH
        if not first:
          pltpu.make_async_copy(
              rows_v.at[buf, j], out_hbm.at[pl.ds(0, CH)], osem.at[buf, j]
          ).wait()
        pltpu.async_copy(
            table_hbm.at[idx_v.at[pl.ds(ioff, CH)]],
            rows_v.at[buf, j],
            gsem.at[buf, j],
        )

    def drain(s, buf):
      for j in range(K):
        ioff = s * SS + j * CH
        pltpu.make_async_copy(
            table_hbm.at[idx_v.at[pl.ds(ioff, CH)]],
            rows_v.at[buf, j],
            gsem.at[buf, j],
        ).wait()

        def scale_body(r, _):
          for u in range(RU):
            for q in range(D // L):
              sl = pl.ds(q * L, L)
              rows_v[buf, j, r * RU + u, sl] = (
                  rows_v[buf, j, r * RU + u, sl] * SCALE
              )
          return 0

        lax.fori_loop(0, CH // RU, scale_body, 0)
        pltpu.async_copy(
            rows_v.at[buf, j],
            out_hbm.at[pl.ds(base + s * SS + j * CH, CH)],
            osem.at[buf, j],
        )

    fire(0, 0, True)
    fire(1, 1, True)

    def loop_body(i, _):
      s = 2 * i
      drain(s, 0)
      fire(s + 2, 0, False)
      drain(s + 1, 1)
      fire(s + 3, 1, False)
      return 0

    lax.fori_loop(0, N_SS // 2 - 1, loop_body, 0)
    drain(N_SS - 2, 0)
    drain(N_SS - 1, 1)
    for buf in range(2):
      for j in range(K):
        pltpu.make_async_copy(
            rows_v.at[buf, j], out_hbm.at[pl.ds(0, CH)], osem.at[buf, j]
        ).wait()

  return k(table, x_flat)


def kernel(table, x):
  x_flat = x.reshape(-1).astype(jnp.int32)
  out = _sc_embed(table, x_flat)
  return out.reshape(x.shape + (D,))


# speedup vs baseline: 1.2751x; 1.0015x over previous
import functools
import math

import jax
import jax.numpy as jnp
from jax import lax
from jax.experimental import pallas as pl
from jax.experimental.pallas import tpu as pltpu
from jax.experimental.pallas import tpu_sc as plsc

VOCAB = 1000000
D = 64
B = 4096 * 200
NC, NS, L = 2, 16, 16
NW = NC * NS
B_PER_W = B // NW
CH = 128
K = 4
SS = K * CH
N_SS = B_PER_W // SS
SCALE = math.sqrt(D)
RU = 8


def _sc_embed(table, x_flat):
  mesh = plsc.VectorSubcoreMesh(core_axis_name="c", subcore_axis_name="s")

  @functools.partial(
      pl.kernel,
      mesh=mesh,
      compiler_params=pltpu.CompilerParams(use_tc_tiling_on_sc=False),
      out_type=jax.ShapeDtypeStruct((B, D), jnp.float32),
      scratch_types=[
          pltpu.VMEM((B_PER_W,), jnp.int32),
          pltpu.VMEM((2, K, CH, D), jnp.float32),
          pltpu.SemaphoreType.DMA((2, K)),
          pltpu.SemaphoreType.DMA((2, K)),
      ],
  )
  def k(table_hbm, idx_hbm, out_hbm, idx_v, rows_v, gsem, osem):
    wid = lax.axis_index("s") * NC + lax.axis_index("c")
    base = wid * B_PER_W

    pltpu.sync_copy(idx_hbm.at[pl.ds(base, B_PER_W)], idx_v)

    def fire(s, buf, first):
      for j in range(K):
        ioff = s * SS + j * CH
        if not first:
          pltpu.make_async_copy(
              rows_v.at[buf, j], out_hbm.at[pl.ds(0, CH)], osem.at[buf, j]
          ).wait()
        pltpu.async_copy(
            table_hbm.at[idx_v.at[pl.ds(ioff, CH)]],
            rows_v.at[buf, j],
            gsem.at[buf, j],
        )

    def drain(s, buf):
      for j in range(K):
        ioff = s * SS + j * CH
        pltpu.make_async_copy(
            table_hbm.at[idx_v.at[pl.ds(ioff, CH)]],
            rows_v.at[buf, j],
            gsem.at[buf, j],
        ).wait()

        @plsc.parallel_loop(0, CH, step=1, unroll=RU)
        def scale_body(r):
          for q in range(D // L):
            sl = pl.ds(q * L, L)
            rows_v[buf, j, r, sl] = rows_v[buf, j, r, sl] * SCALE
        pltpu.async_copy(
            rows_v.at[buf, j],
            out_hbm.at[pl.ds(base + s * SS + j * CH, CH)],
            osem.at[buf, j],
        )

    fire(0, 0, True)
    fire(1, 1, True)

    def loop_body(i, _):
      s = 2 * i
      drain(s, 0)
      fire(s + 2, 0, False)
      drain(s + 1, 1)
      fire(s + 3, 1, False)
      return 0

    lax.fori_loop(0, N_SS // 2 - 1, loop_body, 0)
    drain(N_SS - 2, 0)
    drain(N_SS - 1, 1)
    for buf in range(2):
      for j in range(K):
        pltpu.make_async_copy(
            rows_v.at[buf, j], out_hbm.at[pl.ds(0, CH)], osem.at[buf, j]
        ).wait()

  return k(table, x_flat)


def kernel(table, x):
  x_flat = x.reshape(-1).astype(jnp.int32)
  out = _sc_embed(table, x_flat)
  return out.reshape(x.shape + (D,))
